# Initial kernel scaffold; baseline (speedup 1.0000x reference)
import jax
import jax.numpy as jnp
from jax.experimental import pallas as pl


def kernel(x_in):
    raise NotImplementedError("write your pallas kernel here")


# speedup vs baseline: 37.3924x; 37.3924x over previous
import functools

import numpy as np
import jax
import jax.numpy as jnp
from jax import lax
from jax.experimental import pallas as pl
from jax.experimental.pallas import tpu as pltpu
from jax.experimental.pallas import tpu_sc as plsc

_A8 = np.array([
    [0, 0, 0, 0, 0, 0, 0, 0], [4, 0, 0, 0, 0, 0, 0, 0],
    [2, 2, 2, 2, 0, 0, 0, 0], [-2, 2, 2, 2, 0, 0, 0, 0],
    [2, 2, 0, 0, 2, 2, 0, 0], [-2, 2, 0, 0, 2, 2, 0, 0],
    [2, 2, 0, 0, 0, 0, 2, 2], [-2, 2, 0, 0, 0, 0, 2, 2],
    [2, 0, 2, 0, 2, 0, 2, 0], [-2, 0, 2, 0, 2, 0, 2, 0],
    [2, 0, 2, 0, 0, 2, 0, 2], [-2, 0, 2, 0, 0, 2, 0, 2],
    [2, 0, 0, 2, 2, 0, 0, 2], [-2, 0, 0, 2, 2, 0, 0, 2],
    [2, 0, 0, 2, 0, 2, 2, 0], [-2, 0, 0, 2, 0, 2, 2, 0]], dtype=np.float32)
_T8 = np.array([
    [0, 0, 0, 0, 0, 0, 0, 0], [2, 2, 2, 0, 0, 2, 0, 0],
    [2, 2, 0, 2, 0, 0, 0, 2], [2, 0, 2, 2, 0, 0, 2, 0],
    [0, 2, 2, 2, 2, 0, 0, 0], [2, 2, 0, 0, 2, 0, 2, 0],
    [2, 0, 2, 0, 2, 0, 0, 2], [2, 0, 0, 2, 2, 2, 0, 0],
    [-3, 1, 1, 1, 1, 1, 1, 1], [3, -1, -1, 1, 1, -1, 1, 1],
    [3, -1, 1, -1, 1, 1, 1, -1], [3, 1, -1, -1, 1, 1, -1, 1],
    [3, 1, 1, 1, 1, -1, -1, -1], [3, -1, 1, 1, -1, 1, -1, 1],
    [3, 1, -1, 1, -1, 1, 1, -1], [3, 1, 1, -1, -1, -1, 1, 1]], dtype=np.float32)
_PA = np.repeat(np.arange(16), 16)
_PB = np.tile(np.arange(16), 16)
_AT256 = _A8[_PA] + _T8[_PB]
_ATX_NP = np.ascontiguousarray((-_AT256.T).reshape(-1)).astype(np.float32)

_SQ8 = float(np.sqrt(8.0))
_ASC = float(1.0 / np.sqrt(8.0))
_MAGIC = 12582912.0
_NC, _NS, _L = 2, 16, 16
_NW = _NC * _NS


_GATHER_DNUMS = lax.GatherDimensionNumbers(
    offset_dims=(), collapsed_slice_dims=(0,), start_index_map=(0,))


def _vgather(v, idx):
    return lax.gather(v, idx[:, None], _GATHER_DNUMS, (1,),
                      mode=lax.GatherScatterMode.PROMISE_IN_BOUNDS)


def _lanes_min(v, iot):
    for k in (1, 2, 4, 8):
        v = jnp.minimum(v, _vgather(v, jnp.bitwise_xor(iot, k)))
    return v


def _d8_lanes(z):
    f = [(zi + _MAGIC) - _MAGIC for zi in z]
    d = [z[i] - f[i] for i in range(8)]
    av = [jnp.abs(di) for di in d]
    m = av[0]
    for i in range(1, 8):
        m = jnp.maximum(m, av[i])
    ssum = f[0]
    for i in range(1, 8):
        ssum = ssum + f[i]
    oddf = jnp.bitwise_and(ssum.astype(jnp.int32), 1).astype(jnp.float32)
    q = []
    done = jnp.zeros((_L,), jnp.bool_)
    for i in range(8):
        hit = jnp.logical_and(av[i] == m, jnp.logical_not(done))
        fix = jnp.where(d[i] < 0.0, -1.0, 1.0)
        q.append(f[i] + jnp.where(hit, fix, 0.0) * oddf)
        done = jnp.logical_or(done, hit)
    return q


def _e8_lanes(z):
    q1 = _d8_lanes(z)
    q2m = _d8_lanes([zi - 0.5 for zi in z])
    q2 = [qi + 0.5 for qi in q2m]
    e1 = (z[0] - q1[0]) * (z[0] - q1[0])
    e2 = (z[0] - q2[0]) * (z[0] - q2[0])
    for i in range(1, 8):
        r1 = z[i] - q1[i]
        r2 = z[i] - q2[i]
        e1 = e1 + r1 * r1
        e2 = e2 + r2 * r2
    pick1 = e1 <= e2
    q = [jnp.where(pick1, q1[i], q2[i]) for i in range(8)]
    return q, jnp.where(pick1, e1, e2)


def _make_sc_kernel(n_tok):
    tpw = n_tok // _NW
    mesh = plsc.VectorSubcoreMesh(core_axis_name="c", subcore_axis_name="s")

    @functools.partial(
        pl.kernel,
        out_type=jax.ShapeDtypeStruct((n_tok * 32,), jnp.float32),
        mesh=mesh,
        compiler_params=pltpu.CompilerParams(needs_layout_passes=False),
        scratch_types=[
            pltpu.VMEM((tpw * 24 + 8,), jnp.float32),
            pltpu.VMEM((8 * 256,), jnp.float32),
            pltpu.VMEM((24 * 256,), jnp.float32),
            pltpu.VMEM((3 * 256,), jnp.float32),
            pltpu.VMEM((tpw * 32,), jnp.float32),
        ],
    )
    def sc_quantize(x_hbm, at_hbm, out_hbm, xv, atv, vtab, dtab, outv):
        wid = lax.axis_index("s") * _NC + lax.axis_index("c")
        pltpu.sync_copy(x_hbm.at[pl.ds(wid * (tpw * 24), tpw * 24)],
                        xv.at[pl.ds(0, tpw * 24)])
        pltpu.sync_copy(at_hbm, atv)
        iot = lax.iota(jnp.int32, _L)

        def token_body(tt, carry):
            xrow_lo = xv[pl.ds(tt * 24, _L)]
            xrow_hi = xv[pl.ds(tt * 24 + 8, _L)]
            for c in range(3):
                if c < 2:
                    xs = [xrow_lo[c * 8 + i] for i in range(8)]
                else:
                    xs = [xrow_hi[8 + i] for i in range(8)]
                xb = [jnp.full((_L,), xs[i] * _SQ8, jnp.float32) for i in range(8)]

                def group_body(g, _, c=c, xb=xb):
                    at_ = [atv[pl.ds(i * 256 + g * 16, _L)] for i in range(8)]
                    z = [(xb[i] + at_[i]) * 0.25 for i in range(8)]
                    q, e = _e8_lanes(z)
                    for i in range(8):
                        vtab[pl.ds((c * 8 + i) * 256 + g * 16, _L)] = (
                            (4.0 * q[i] - at_[i]) * _ASC)
                    dtab[pl.ds(c * 256 + g * 16, _L)] = e
                    return 0

                lax.fori_loop(0, 16, group_body, 0)

            def a_body(a, carry):
                d1a = dtab[pl.ds(a * 16, _L)]

                def b_body(b, carry2):
                    m2, mab2 = carry2
                    d2b = dtab[pl.ds(256 + b * 16, _L)]
                    cx = lax.bitwise_xor(a, b)
                    d3c = dtab[pl.ds(512 + cx * 16, _L)]
                    s = d1a + d2b + d3c
                    upd = s < m2
                    abv = jnp.full((_L,), a * 16 + b, jnp.int32)
                    return (jnp.where(upd, s, m2), jnp.where(upd, abv, mab2))

                return lax.fori_loop(0, 16, b_body, carry)

            m0 = jnp.full((_L,), 1e30, jnp.float32)
            mab0 = jnp.zeros((_L,), jnp.int32)
            m, mab = lax.fori_loop(0, 16, a_body, (m0, mab0))

            minv = _lanes_min(m, iot)
            tlane = _lanes_min(jnp.where(m == minv, iot, 999), iot)
            ab = _lanes_min(jnp.where(iot == tlane, mab, 1 << 30), iot)
            a_w = lax.shift_right_logical(ab, 4)
            b_w = jnp.bitwise_and(ab, 15)
            c_w = lax.bitwise_xor(a_w, b_w)
            j1 = a_w * 16 + tlane
            j2 = b_w * 16 + tlane
            j3 = c_w * 16 + tlane

            idx1 = iot * 256 + jnp.where(iot < 8, j1, j2)
            outv[pl.ds(tt * 32, _L)] = plsc.load_gather(vtab, [idx1])
            idx2 = jnp.where(iot < 8, (iot + 16) * 256 + j3, 0)
            outv[pl.ds(tt * 32 + 16, _L)] = plsc.load_gather(vtab, [idx2])
            return carry

        lax.fori_loop(0, tpw, token_body, 0)
        pltpu.sync_copy(outv, out_hbm.at[pl.ds(wid * (tpw * 32), tpw * 32)])

    return sc_quantize


def kernel(x_in):
    n_tok = x_in.shape[0]
    xflat = jnp.reshape(x_in.astype(jnp.float32), (-1,))
    out = _make_sc_kernel(n_tok)(xflat, jnp.asarray(_ATX_NP))
    return out.reshape(n_tok, 32)[:, :24]


# speedup vs baseline: 58.6588x; 1.5687x over previous
import functools

import numpy as np
import jax
import jax.numpy as jnp
from jax import lax
from jax.experimental import pallas as pl
from jax.experimental.pallas import tpu as pltpu
from jax.experimental.pallas import tpu_sc as plsc

_A8 = np.array([
    [0, 0, 0, 0, 0, 0, 0, 0], [4, 0, 0, 0, 0, 0, 0, 0],
    [2, 2, 2, 2, 0, 0, 0, 0], [-2, 2, 2, 2, 0, 0, 0, 0],
    [2, 2, 0, 0, 2, 2, 0, 0], [-2, 2, 0, 0, 2, 2, 0, 0],
    [2, 2, 0, 0, 0, 0, 2, 2], [-2, 2, 0, 0, 0, 0, 2, 2],
    [2, 0, 2, 0, 2, 0, 2, 0], [-2, 0, 2, 0, 2, 0, 2, 0],
    [2, 0, 2, 0, 0, 2, 0, 2], [-2, 0, 2, 0, 0, 2, 0, 2],
    [2, 0, 0, 2, 2, 0, 0, 2], [-2, 0, 0, 2, 2, 0, 0, 2],
    [2, 0, 0, 2, 0, 2, 2, 0], [-2, 0, 0, 2, 0, 2, 2, 0]], dtype=np.float32)
_T8 = np.array([
    [0, 0, 0, 0, 0, 0, 0, 0], [2, 2, 2, 0, 0, 2, 0, 0],
    [2, 2, 0, 2, 0, 0, 0, 2], [2, 0, 2, 2, 0, 0, 2, 0],
    [0, 2, 2, 2, 2, 0, 0, 0], [2, 2, 0, 0, 2, 0, 2, 0],
    [2, 0, 2, 0, 2, 0, 0, 2], [2, 0, 0, 2, 2, 2, 0, 0],
    [-3, 1, 1, 1, 1, 1, 1, 1], [3, -1, -1, 1, 1, -1, 1, 1],
    [3, -1, 1, -1, 1, 1, 1, -1], [3, 1, -1, -1, 1, 1, -1, 1],
    [3, 1, 1, 1, 1, -1, -1, -1], [3, -1, 1, 1, -1, 1, -1, 1],
    [3, 1, -1, 1, -1, 1, 1, -1], [3, 1, 1, -1, -1, -1, 1, 1]], dtype=np.float32)
_PA = np.repeat(np.arange(16), 16)
_PB = np.tile(np.arange(16), 16)
_AT256 = _A8[_PA] + _T8[_PB]
_ATX_NP = np.ascontiguousarray((-_AT256.T).reshape(-1)).astype(np.float32)

_SQ8 = float(np.sqrt(8.0))
_ASC = float(1.0 / np.sqrt(8.0))
_MAGIC = 12582912.0
_NC, _NS, _L = 2, 16, 16
_NW = _NC * _NS

_GATHER_DNUMS = lax.GatherDimensionNumbers(
    offset_dims=(), collapsed_slice_dims=(0,), start_index_map=(0,))


def _vgather(v, idx):
    return lax.gather(v, idx[:, None], _GATHER_DNUMS, (1,),
                      mode=lax.GatherScatterMode.PROMISE_IN_BOUNDS)


def _lanes_min(v, iot):
    for k in (1, 2, 4, 8):
        v = jnp.minimum(v, _vgather(v, jnp.bitwise_xor(iot, k)))
    return v


def _d8_err(z):
    f = [(zi + _MAGIC) - _MAGIC for zi in z]
    d = [z[i] - f[i] for i in range(8)]
    s2 = d[0] * d[0]
    ssum = f[0]
    m = jnp.abs(d[0])
    for i in range(1, 8):
        s2 = s2 + d[i] * d[i]
        ssum = ssum + f[i]
        m = jnp.maximum(m, jnp.abs(d[i]))
    oddf = jnp.bitwise_and(ssum.astype(jnp.int32), 1).astype(jnp.float32)
    return s2 + oddf * (1.0 - (m + m))


def _d8_full(z):
    f = [(zi + _MAGIC) - _MAGIC for zi in z]
    d = [z[i] - f[i] for i in range(8)]
    av = [jnp.abs(di) for di in d]
    m = av[0]
    for i in range(1, 8):
        m = jnp.maximum(m, av[i])
    ssum = f[0]
    for i in range(1, 8):
        ssum = ssum + f[i]
    oddf = jnp.bitwise_and(ssum.astype(jnp.int32), 1).astype(jnp.float32)
    q = []
    done = jnp.zeros((_L,), jnp.bool_)
    for i in range(8):
        hit = jnp.logical_and(av[i] == m, jnp.logical_not(done))
        fix = jnp.where(d[i] < 0.0, -1.0, 1.0)
        q.append(f[i] + jnp.where(hit, fix, 0.0) * oddf)
        done = jnp.logical_or(done, hit)
    return q


def _e8_full(z):
    q1 = _d8_full(z)
    q2m = _d8_full([zi - 0.5 for zi in z])
    q2 = [qi + 0.5 for qi in q2m]
    e1 = (z[0] - q1[0]) * (z[0] - q1[0])
    e2 = (z[0] - q2[0]) * (z[0] - q2[0])
    for i in range(1, 8):
        r1 = z[i] - q1[i]
        r2 = z[i] - q2[i]
        e1 = e1 + r1 * r1
        e2 = e2 + r2 * r2
    pick1 = e1 <= e2
    return [jnp.where(pick1, q1[i], q2[i]) for i in range(8)]


def _make_sc_kernel(n_tok):
    tpw = n_tok // _NW
    npairs = tpw * 3
    assert npairs % _L == 0
    mesh = plsc.VectorSubcoreMesh(core_axis_name="c", subcore_axis_name="s")

    @functools.partial(
        pl.kernel,
        out_type=jax.ShapeDtypeStruct((n_tok * 32,), jnp.float32),
        mesh=mesh,
        compiler_params=pltpu.CompilerParams(needs_layout_passes=False),
        scratch_types=[
            pltpu.VMEM((tpw * 24 + 8,), jnp.float32),
            pltpu.VMEM((8 * 256,), jnp.float32),
            pltpu.VMEM((3 * 256,), jnp.float32),
            pltpu.VMEM((tpw * 16,), jnp.int32),
            pltpu.VMEM((tpw * 32,), jnp.float32),
        ],
    )
    def sc_quantize(x_hbm, at_hbm, out_hbm, xv, atv, dtab, jtab, outv):
        wid = lax.axis_index("s") * _NC + lax.axis_index("c")
        pltpu.sync_copy(x_hbm.at[pl.ds(wid * (tpw * 24), tpw * 24)],
                        xv.at[pl.ds(0, tpw * 24)])
        pltpu.sync_copy(at_hbm, atv)
        iot = lax.iota(jnp.int32, _L)

        def token_body(tt, carry):
            xrow_lo = xv[pl.ds(tt * 24, _L)]
            xrow_hi = xv[pl.ds(tt * 24 + 8, _L)]
            xb = []
            for c in range(3):
                xs = ([xrow_lo[c * 8 + i] for i in range(8)] if c < 2
                      else [xrow_hi[8 + i] for i in range(8)])
                xb.append([jnp.full((_L,), xs[i] * _SQ8, jnp.float32)
                           for i in range(8)])

            def group_body(g, _):
                at_ = [atv[pl.ds(i * 256 + g * 16, _L)] for i in range(8)]
                for c in range(3):
                    z = [(xb[c][i] + at_[i]) * 0.25 for i in range(8)]
                    e1 = _d8_err(z)
                    e2 = _d8_err([zi - 0.5 for zi in z])
                    dtab[pl.ds(c * 256 + g * 16, _L)] = jnp.minimum(e1, e2)
                return 0

            lax.fori_loop(0, 16, group_body, 0)

            def a_body(a, carry):
                d1a = dtab[pl.ds(a * 16, _L)]

                def b_body(b, carry2):
                    m2, mab2 = carry2
                    d2b = dtab[pl.ds(256 + b * 16, _L)]
                    cx = lax.bitwise_xor(a, b)
                    d3c = dtab[pl.ds(512 + cx * 16, _L)]
                    s = d1a + d2b + d3c
                    upd = s < m2
                    abv = jnp.full((_L,), a * 16 + b, jnp.int32)
                    return (jnp.where(upd, s, m2), jnp.where(upd, abv, mab2))

                return lax.fori_loop(0, 16, b_body, carry)

            m0 = jnp.full((_L,), 1e30, jnp.float32)
            mab0 = jnp.zeros((_L,), jnp.int32)
            m, mab = lax.fori_loop(0, 16, a_body, (m0, mab0))

            minv = _lanes_min(m, iot)
            tlane = _lanes_min(jnp.where(m == minv, iot, 999), iot)
            ab = _lanes_min(jnp.where(iot == tlane, mab, 1 << 30), iot)
            a_w = lax.shift_right_logical(ab, 4)
            b_w = jnp.bitwise_and(ab, 15)
            c_w = lax.bitwise_xor(a_w, b_w)
            jv = jnp.where(iot == 0, a_w * 16 + tlane,
                           jnp.where(iot == 1, b_w * 16 + tlane,
                                     c_w * 16 + tlane))
            jtab[pl.ds(tt * 16, _L)] = jv
            return carry

        lax.fori_loop(0, tpw, token_body, 0)

        for g in range(npairs // _L):
            p = iot + g * _L
            toks = lax.div(p, 3)
            chks = lax.rem(p, 3)
            j = plsc.load_gather(jtab, [toks * 16 + chks])
            z = []
            atg = []
            for i in range(8):
                xg = plsc.load_gather(xv, [toks * 24 + chks * 8 + i])
                ai = plsc.load_gather(atv, [i * 256 + j])
                atg.append(ai)
                z.append((xg * _SQ8 + ai) * 0.25)
            q = _e8_full(z)
            for i in range(8):
                plsc.store_scatter(outv, [toks * 32 + chks * 8 + i],
                                   (4.0 * q[i] - atg[i]) * _ASC)

        pltpu.sync_copy(outv, out_hbm.at[pl.ds(wid * (tpw * 32), tpw * 32)])

    return sc_quantize


def kernel(x_in):
    n_tok = x_in.shape[0]
    xflat = jnp.reshape(x_in.astype(jnp.float32), (-1,))
    out = _make_sc_kernel(n_tok)(xflat, jnp.asarray(_ATX_NP))
    return out.reshape(n_tok, 32)[:, :24]


# speedup vs baseline: 72.9746x; 1.2441x over previous
import functools

import numpy as np
import jax
import jax.numpy as jnp
from jax import lax
from jax.experimental import pallas as pl
from jax.experimental.pallas import tpu as pltpu
from jax.experimental.pallas import tpu_sc as plsc

_A8 = np.array([
    [0, 0, 0, 0, 0, 0, 0, 0], [4, 0, 0, 0, 0, 0, 0, 0],
    [2, 2, 2, 2, 0, 0, 0, 0], [-2, 2, 2, 2, 0, 0, 0, 0],
    [2, 2, 0, 0, 2, 2, 0, 0], [-2, 2, 0, 0, 2, 2, 0, 0],
    [2, 2, 0, 0, 0, 0, 2, 2], [-2, 2, 0, 0, 0, 0, 2, 2],
    [2, 0, 2, 0, 2, 0, 2, 0], [-2, 0, 2, 0, 2, 0, 2, 0],
    [2, 0, 2, 0, 0, 2, 0, 2], [-2, 0, 2, 0, 0, 2, 0, 2],
    [2, 0, 0, 2, 2, 0, 0, 2], [-2, 0, 0, 2, 2, 0, 0, 2],
    [2, 0, 0, 2, 0, 2, 2, 0], [-2, 0, 0, 2, 0, 2, 2, 0]], dtype=np.float32)
_T8 = np.array([
    [0, 0, 0, 0, 0, 0, 0, 0], [2, 2, 2, 0, 0, 2, 0, 0],
    [2, 2, 0, 2, 0, 0, 0, 2], [2, 0, 2, 2, 0, 0, 2, 0],
    [0, 2, 2, 2, 2, 0, 0, 0], [2, 2, 0, 0, 2, 0, 2, 0],
    [2, 0, 2, 0, 2, 0, 0, 2], [2, 0, 0, 2, 2, 2, 0, 0],
    [-3, 1, 1, 1, 1, 1, 1, 1], [3, -1, -1, 1, 1, -1, 1, 1],
    [3, -1, 1, -1, 1, 1, 1, -1], [3, 1, -1, -1, 1, 1, -1, 1],
    [3, 1, 1, 1, 1, -1, -1, -1], [3, -1, 1, 1, -1, 1, -1, 1],
    [3, 1, -1, 1, -1, 1, 1, -1], [3, 1, 1, -1, -1, -1, 1, 1]], dtype=np.float32)
_PA = np.repeat(np.arange(16), 16)
_PB = np.tile(np.arange(16), 16)
_AT256 = _A8[_PA] + _T8[_PB]
_ATX_NP = np.ascontiguousarray((-_AT256.T).reshape(-1)).astype(np.float32)

_SQ8 = float(np.sqrt(8.0))
_ASC = float(1.0 / np.sqrt(8.0))
_MAGIC = 12582912.0
_NC, _NS, _L = 2, 16, 16
_NW = _NC * _NS

_GATHER_DNUMS = lax.GatherDimensionNumbers(
    offset_dims=(), collapsed_slice_dims=(0,), start_index_map=(0,))


def _vgather(v, idx):
    return lax.gather(v, idx[:, None], _GATHER_DNUMS, (1,),
                      mode=lax.GatherScatterMode.PROMISE_IN_BOUNDS)


def _lanes_min(v, iot):
    for k in (1, 2, 4, 8):
        v = jnp.minimum(v, _vgather(v, jnp.bitwise_xor(iot, k)))
    return v


def _d8_err(z):
    f = [(zi + _MAGIC) - _MAGIC for zi in z]
    d = [z[i] - f[i] for i in range(8)]
    s2 = d[0] * d[0]
    ssum = f[0]
    m = jnp.abs(d[0])
    for i in range(1, 8):
        s2 = s2 + d[i] * d[i]
        ssum = ssum + f[i]
        m = jnp.maximum(m, jnp.abs(d[i]))
    oddf = jnp.bitwise_and(ssum.astype(jnp.int32), 1).astype(jnp.float32)
    return s2 + oddf * (1.0 - (m + m))


def _d8_full(z):
    f = [(zi + _MAGIC) - _MAGIC for zi in z]
    d = [z[i] - f[i] for i in range(8)]
    av = [jnp.abs(di) for di in d]
    m = av[0]
    for i in range(1, 8):
        m = jnp.maximum(m, av[i])
    ssum = f[0]
    for i in range(1, 8):
        ssum = ssum + f[i]
    oddf = jnp.bitwise_and(ssum.astype(jnp.int32), 1).astype(jnp.float32)
    q = []
    done = jnp.zeros((_L,), jnp.bool_)
    for i in range(8):
        hit = jnp.logical_and(av[i] == m, jnp.logical_not(done))
        fix = jnp.where(d[i] < 0.0, -1.0, 1.0)
        q.append(f[i] + jnp.where(hit, fix, 0.0) * oddf)
        done = jnp.logical_or(done, hit)
    return q


def _e8_full(z):
    q1 = _d8_full(z)
    q2m = _d8_full([zi - 0.5 for zi in z])
    q2 = [qi + 0.5 for qi in q2m]
    e1 = (z[0] - q1[0]) * (z[0] - q1[0])
    e2 = (z[0] - q2[0]) * (z[0] - q2[0])
    for i in range(1, 8):
        r1 = z[i] - q1[i]
        r2 = z[i] - q2[i]
        e1 = e1 + r1 * r1
        e2 = e2 + r2 * r2
    pick1 = e1 <= e2
    return [jnp.where(pick1, q1[i], q2[i]) for i in range(8)]


def _make_sc_kernel(n_tok):
    tpw = n_tok // _NW
    npairs = tpw * 3
    assert npairs % _L == 0
    mesh = plsc.VectorSubcoreMesh(core_axis_name="c", subcore_axis_name="s")

    @functools.partial(
        pl.kernel,
        out_type=jax.ShapeDtypeStruct((n_tok * 32,), jnp.float32),
        mesh=mesh,
        compiler_params=pltpu.CompilerParams(needs_layout_passes=False),
        scratch_types=[
            pltpu.VMEM((tpw * 24 + 8,), jnp.float32),
            pltpu.VMEM((8 * 256,), jnp.float32),
            pltpu.VMEM((3 * 256,), jnp.float32),
            pltpu.VMEM((tpw * 16,), jnp.int32),
            pltpu.VMEM((tpw * 32,), jnp.float32),
        ],
    )
    def sc_quantize(x_hbm, at_hbm, out_hbm, xv, atv, dtab, jtab, outv):
        wid = lax.axis_index("s") * _NC + lax.axis_index("c")
        pltpu.sync_copy(x_hbm.at[pl.ds(wid * (tpw * 24), tpw * 24)],
                        xv.at[pl.ds(0, tpw * 24)])
        pltpu.sync_copy(at_hbm, atv)
        iot = lax.iota(jnp.int32, _L)

        def token_body(tt, carry):
            xrow_lo = xv[pl.ds(tt * 24, _L)]
            xrow_hi = xv[pl.ds(tt * 24 + 8, _L)]
            xb = []
            for c in range(3):
                xs = ([xrow_lo[c * 8 + i] for i in range(8)] if c < 2
                      else [xrow_hi[8 + i] for i in range(8)])
                xb.append([jnp.full((_L,), xs[i] * _SQ8, jnp.float32)
                           for i in range(8)])

            def group_body(g, _):
                at_ = [atv[pl.ds(i * 256 + g * 16, _L)] for i in range(8)]
                for c in range(3):
                    z = [(xb[c][i] + at_[i]) * 0.25 for i in range(8)]
                    e1 = _d8_err(z)
                    e2 = _d8_err([zi - 0.5 for zi in z])
                    dtab[pl.ds(c * 256 + g * 16, _L)] = jnp.minimum(e1, e2)
                return 0

            lax.fori_loop(0, 16, group_body, 0)

            d1v = [dtab[pl.ds(a * 16, _L)] for a in range(16)]
            d2v = [dtab[pl.ds(256 + b * 16, _L)] for b in range(16)]
            d3v = [dtab[pl.ds(512 + c * 16, _L)] for c in range(16)]
            accs = []
            for blk in range(4):
                m = jnp.full((_L,), 1e30, jnp.float32)
                mab = jnp.zeros((_L,), jnp.int32)
                for a in range(blk * 4, blk * 4 + 4):
                    for b in range(16):
                        s = d1v[a] + d2v[b] + d3v[a ^ b]
                        upd = s < m
                        m = jnp.where(upd, s, m)
                        mab = jnp.where(upd, jnp.full((_L,), a * 16 + b,
                                                      jnp.int32), mab)
                accs.append((m, mab))
            m, mab = accs[0]
            for m2, mab2 in accs[1:]:
                upd = m2 < m
                m = jnp.where(upd, m2, m)
                mab = jnp.where(upd, mab2, mab)

            minv = _lanes_min(m, iot)
            tlane = _lanes_min(jnp.where(m == minv, iot, 999), iot)
            ab = _lanes_min(jnp.where(iot == tlane, mab, 1 << 30), iot)
            a_w = lax.shift_right_logical(ab, 4)
            b_w = jnp.bitwise_and(ab, 15)
            c_w = lax.bitwise_xor(a_w, b_w)
            jv = jnp.where(iot == 0, a_w * 16 + tlane,
                           jnp.where(iot == 1, b_w * 16 + tlane,
                                     c_w * 16 + tlane))
            jtab[pl.ds(tt * 16, _L)] = jv
            return carry

        lax.fori_loop(0, tpw, token_body, 0)

        for g in range(npairs // _L):
            p = iot + g * _L
            toks = lax.div(p, 3)
            chks = lax.rem(p, 3)
            j = plsc.load_gather(jtab, [toks * 16 + chks])
            z = []
            atg = []
            for i in range(8):
                xg = plsc.load_gather(xv, [toks * 24 + chks * 8 + i])
                ai = plsc.load_gather(atv, [i * 256 + j])
                atg.append(ai)
                z.append((xg * _SQ8 + ai) * 0.25)
            q = _e8_full(z)
            for i in range(8):
                plsc.store_scatter(outv, [toks * 32 + chks * 8 + i],
                                   (4.0 * q[i] - atg[i]) * _ASC)

        pltpu.sync_copy(outv, out_hbm.at[pl.ds(wid * (tpw * 32), tpw * 32)])

    return sc_quantize


def kernel(x_in):
    n_tok = x_in.shape[0]
    xflat = jnp.reshape(x_in.astype(jnp.float32), (-1,))
    out = _make_sc_kernel(n_tok)(xflat, jnp.asarray(_ATX_NP))
    return out.reshape(n_tok, 32)[:, :24]
